# baseline (speedup 1.0000x reference)
import functools

import jax
import jax.numpy as jnp
import numpy as np
from jax import lax
from jax.experimental import pallas as pl
from jax.experimental.pallas import tpu as pltpu
from jax.experimental.pallas import tpu_sc as plsc

N_NODES = 10000
N_EDGES = 320000
D = 128

NUM_CORES = 2
NUM_SUBCORES = 16
NUM_WORKERS = NUM_CORES * NUM_SUBCORES

CHUNK = 128
GRP = 16
CHUNKS_PER_WORKER = 80
EDGES_PER_WORKER = CHUNKS_PER_WORKER * CHUNK
E_PAD = EDGES_PER_WORKER * NUM_WORKERS

ACC_ROWS = 10240
N_DUMMY = ACC_ROWS - N_NODES
ZERO_ROWS = ACC_ROWS // NUM_SUBCORES
OUT_ROWS_PER_TILE = ACC_ROWS // NUM_SUBCORES

_PAD = E_PAD - N_EDGES
_AR = np.arange(_PAD, dtype=np.int32)
_PAD_ROW = (N_NODES + (_AR % N_DUMMY)).reshape(_PAD // CHUNK, CHUNK)
_PAD_COL = ((_AR * 37) % N_NODES).reshape(_PAD // CHUNK, CHUNK)


def _matmul_body(x_ref, w_ref, b_ref, h_ref):
    h_ref[...] = lax.dot_general(
        x_ref[...], w_ref[...], (((1,), (1,)), ((), ())),
        preferred_element_type=jnp.float32,
    ) + b_ref[...]


def _linear(x, W_w, W_b):
    return pl.pallas_call(
        _matmul_body,
        grid=(5,),
        in_specs=[
            pl.BlockSpec((2000, D), lambda i: (i, 0)),
            pl.BlockSpec((D, D), lambda i: (0, 0)),
            pl.BlockSpec((1, D), lambda i: (0, 0)),
        ],
        out_specs=pl.BlockSpec((2000, D), lambda i: (i, 0)),
        out_shape=jax.ShapeDtypeStruct((N_NODES, D), jnp.float32),
    )(x, W_w, W_b.reshape(1, D))


def _combine_body(p_ref, o_ref):
    o_ref[...] = p_ref[0] + p_ref[1]


def _combine(partials):
    return pl.pallas_call(
        _combine_body,
        grid=(10,),
        in_specs=[pl.BlockSpec((NUM_CORES, 1000, D), lambda i: (0, i, 0))],
        out_specs=pl.BlockSpec((1000, D), lambda i: (i, 0)),
        out_shape=jax.ShapeDtypeStruct((N_NODES, D), jnp.float32),
    )(partials)


@functools.partial(
    pl.kernel,
    mesh=plsc.VectorSubcoreMesh(core_axis_name="c", subcore_axis_name="s"),
    out_type=jax.ShapeDtypeStruct((NUM_CORES, ACC_ROWS, D), jnp.float32),
    scratch_types=[
        pltpu.VMEM((2, CHUNK, D), jnp.float32),
        pltpu.VMEM((GRP, CHUNK), jnp.int32),
        pltpu.VMEM((GRP, CHUNK), jnp.int32),
        pltpu.VMEM_SHARED((ACC_ROWS, D), jnp.float32),
        pltpu.SemaphoreType.DMA,
        pltpu.SemaphoreType.DMA,
        pltpu.SemaphoreType.DMA,
        pltpu.SemaphoreType.DMA,
    ],
)
def _sc_segment_sum(h_hbm, col_hbm, row_hbm, out_hbm,
                    buf_v, col_v, row_v, acc_sh, gsem0, gsem1, ssem0, ssem1):
    cid = lax.axis_index("c")
    sid = lax.axis_index("s")
    wid = cid * NUM_SUBCORES + sid
    gsems = (gsem0, gsem1)
    ssems = (ssem0, ssem1)

    zbuf = buf_v.at[0]

    def _zrow(i, _):
        for c in range(D // 16):
            zbuf[i, pl.ds(c * 16, 16)] = jnp.zeros((16,), jnp.float32)
        return 0
    lax.fori_loop(0, CHUNK, _zrow, 0)
    for r in range(ZERO_ROWS // CHUNK):
        pltpu.sync_copy(
            zbuf, acc_sh.at[pl.ds(sid * ZERO_ROWS + r * CHUNK, CHUNK)])

    plsc.subcore_barrier()

    base = wid * CHUNKS_PER_WORKER

    def _group(g, _):
        off = pl.multiple_of(base + g * GRP, 8)
        pltpu.sync_copy(col_hbm.at[pl.ds(off, GRP)], col_v)
        pltpu.sync_copy(row_hbm.at[pl.ds(off, GRP)], row_v)

        gcp = [None, None]
        scp = [None, None]

        def _gather(k):
            gcp[k % 2] = pltpu.async_copy(
                h_hbm.at[col_v.at[k]], buf_v.at[k % 2], gsems[k % 2])

        _gather(0)
        for k in range(GRP):
            b = k % 2
            gcp[b].wait()
            scp[b] = pltpu.async_copy(
                buf_v.at[b], acc_sh.at[row_v.at[k]], ssems[b], add=True)
            if k + 1 < GRP:
                if scp[1 - b] is not None:
                    scp[1 - b].wait()
                _gather(k + 1)
        scp[(GRP - 2) % 2].wait()
        scp[(GRP - 1) % 2].wait()
        return 0
    lax.fori_loop(0, CHUNKS_PER_WORKER // GRP, _group, 0)

    plsc.subcore_barrier()

    pltpu.sync_copy(
        acc_sh.at[pl.ds(sid * OUT_ROWS_PER_TILE, OUT_ROWS_PER_TILE)],
        out_hbm.at[cid, pl.ds(sid * OUT_ROWS_PER_TILE, OUT_ROWS_PER_TILE)],
    )


def kernel(x, edge_index, W_w, W_b, a_w, a_b):
    h = _linear(x, W_w, W_b)

    ei = edge_index.astype(jnp.int32).reshape(2, N_EDGES // CHUNK, CHUNK)
    row2d = jnp.concatenate([ei[0], jnp.asarray(_PAD_ROW)], axis=0)
    col2d = jnp.concatenate([ei[1], jnp.asarray(_PAD_COL)], axis=0)

    partials = _sc_segment_sum(h, col2d, row2d)
    return _combine(partials)


# speedup vs baseline: 1.1634x; 1.1634x over previous
import functools

import jax
import jax.numpy as jnp
from jax import lax
from jax.experimental import pallas as pl
from jax.experimental.pallas import tpu as pltpu
from jax.experimental.pallas import tpu_sc as plsc

N_NODES = 10000
N_EDGES = 320000
D = 128

NUM_CORES = 2
NUM_SUBCORES = 16
NUM_WORKERS = NUM_CORES * NUM_SUBCORES

CHUNK = 128
GRP = 8
N_CHUNKS = N_EDGES // CHUNK
N_GROUPS = N_CHUNKS // GRP
TAIL_CHUNKS = N_CHUNKS - N_GROUPS * GRP
BIG_WORKERS = N_GROUPS % NUM_WORKERS
GRP_BIG = N_GROUPS // NUM_WORKERS + 1
GRP_SMALL = N_GROUPS // NUM_WORKERS

ACC_ROWS = 10240
ZERO_ROWS = ACC_ROWS // NUM_SUBCORES
OUT_ROWS_PER_TILE = ACC_ROWS // NUM_SUBCORES


def _matmul_body(x_ref, w_ref, b_ref, h_ref):
    h_ref[...] = lax.dot_general(
        x_ref[...], w_ref[...], (((1,), (1,)), ((), ())),
        preferred_element_type=jnp.float32,
    ) + b_ref[...]


def _linear(x, W_w, W_b):
    return pl.pallas_call(
        _matmul_body,
        grid=(5,),
        in_specs=[
            pl.BlockSpec((2000, D), lambda i: (i, 0)),
            pl.BlockSpec((D, D), lambda i: (0, 0)),
            pl.BlockSpec((1, D), lambda i: (0, 0)),
        ],
        out_specs=pl.BlockSpec((2000, D), lambda i: (i, 0)),
        out_shape=jax.ShapeDtypeStruct((N_NODES, D), jnp.float32),
    )(x, W_w, W_b.reshape(1, D))


def _combine_body(p_ref, o_ref):
    o_ref[...] = p_ref[0] + p_ref[1]


def _combine(partials):
    return pl.pallas_call(
        _combine_body,
        grid=(10,),
        in_specs=[pl.BlockSpec((NUM_CORES, 1000, D), lambda i: (0, i, 0))],
        out_specs=pl.BlockSpec((1000, D), lambda i: (i, 0)),
        out_shape=jax.ShapeDtypeStruct((N_NODES, D), jnp.float32),
    )(partials)


@functools.partial(
    pl.kernel,
    mesh=plsc.VectorSubcoreMesh(core_axis_name="c", subcore_axis_name="s"),
    out_type=jax.ShapeDtypeStruct((NUM_CORES, ACC_ROWS, D), jnp.float32),
    scratch_types=[
        pltpu.VMEM((2, CHUNK, D), jnp.float32),
        pltpu.VMEM((GRP * CHUNK,), jnp.int32),
        pltpu.VMEM((GRP * CHUNK,), jnp.int32),
        pltpu.VMEM((GRP, CHUNK), jnp.int32),
        pltpu.VMEM((GRP, CHUNK), jnp.int32),
        pltpu.VMEM_SHARED((ACC_ROWS, D), jnp.float32),
        pltpu.SemaphoreType.DMA,
        pltpu.SemaphoreType.DMA,
    ],
)
def _sc_segment_sum(h_hbm, ei_hbm, out_hbm,
                    buf_v, col1_v, row1_v, col_v, row_v, acc_sh, sem0, sem1):
    cid = lax.axis_index("c")
    sid = lax.axis_index("s")
    wid = cid * NUM_SUBCORES + sid
    sems = (sem0, sem1)

    zbuf = buf_v.at[0]

    def _zrow(i, _):
        for c in range(D // 16):
            zbuf[i, pl.ds(c * 16, 16)] = jnp.zeros((16,), jnp.float32)
        return 0
    lax.fori_loop(0, CHUNK, _zrow, 0)
    for r in range(ZERO_ROWS // CHUNK):
        pltpu.sync_copy(
            zbuf, acc_sh.at[pl.ds(sid * ZERO_ROWS + r * CHUNK, CHUNK)])

    plsc.subcore_barrier()

    n_groups = jnp.where(wid < BIG_WORKERS, GRP_BIG, GRP_SMALL)
    group_start = jnp.where(
        wid < BIG_WORKERS,
        wid * GRP_BIG,
        BIG_WORKERS * GRP_BIG + (wid - BIG_WORKERS) * GRP_SMALL,
    )

    def _stage(first_chunk, n_chunks):
        off = pl.multiple_of(first_chunk * CHUNK, 8)
        pltpu.sync_copy(ei_hbm.at[1, pl.ds(off, n_chunks * CHUNK)],
                        col1_v.at[pl.ds(0, n_chunks * CHUNK)])
        pltpu.sync_copy(ei_hbm.at[0, pl.ds(off, n_chunks * CHUNK)],
                        row1_v.at[pl.ds(0, n_chunks * CHUNK)])
        for k in range(n_chunks):
            for c in range(CHUNK // 16):
                col_v[k, pl.ds(c * 16, 16)] = col1_v[pl.ds(k * CHUNK + c * 16, 16)]
                row_v[k, pl.ds(c * 16, 16)] = row1_v[pl.ds(k * CHUNK + c * 16, 16)]

    def _run_chunks(n_chunks):
        copies = [None, None]

        def _gather(k):
            copies[k % 2] = pltpu.async_copy(
                h_hbm.at[col_v.at[k]], buf_v.at[k % 2], sems[k % 2])

        _gather(0)
        for k in range(n_chunks):
            if k + 1 < n_chunks:
                _gather(k + 1)
            copies[k % 2].wait()
            pltpu.sync_copy(buf_v.at[k % 2], acc_sh.at[row_v.at[k]], add=True)

    def _group(g, _):
        _stage((group_start + g) * GRP, GRP)
        _run_chunks(GRP)
        return 0
    lax.fori_loop(0, n_groups, _group, 0)

    @pl.when(wid == NUM_WORKERS - 1)
    def _tail():
        _stage(jnp.int32(N_GROUPS * GRP), TAIL_CHUNKS)
        _run_chunks(TAIL_CHUNKS)

    plsc.subcore_barrier()

    pltpu.sync_copy(
        acc_sh.at[pl.ds(sid * OUT_ROWS_PER_TILE, OUT_ROWS_PER_TILE)],
        out_hbm.at[cid, pl.ds(sid * OUT_ROWS_PER_TILE, OUT_ROWS_PER_TILE)],
    )


def kernel(x, edge_index, W_w, W_b, a_w, a_b):
    h = _linear(x, W_w, W_b)
    partials = _sc_segment_sum(h, edge_index.astype(jnp.int32))
    return _combine(partials)


# speedup vs baseline: 1.2733x; 1.0944x over previous
import functools

import jax
import jax.numpy as jnp
from jax import lax
from jax.experimental import pallas as pl
from jax.experimental.pallas import tpu as pltpu
from jax.experimental.pallas import tpu_sc as plsc

N_NODES = 10000
N_EDGES = 320000
D = 128

NUM_CORES = 2
NUM_SUBCORES = 16
NUM_WORKERS = NUM_CORES * NUM_SUBCORES

CHUNK = 128
GRP = 8
N_CHUNKS = N_EDGES // CHUNK
N_GROUPS = N_CHUNKS // GRP
TAIL_CHUNKS = N_CHUNKS - N_GROUPS * GRP
BIG_WORKERS = N_GROUPS % NUM_WORKERS
GRP_BIG = N_GROUPS // NUM_WORKERS + 1
GRP_SMALL = N_GROUPS // NUM_WORKERS

ACC_ROWS = 10240
ZERO_ROWS = ACC_ROWS // NUM_SUBCORES
OUT_ROWS_PER_TILE = ACC_ROWS // NUM_SUBCORES


def _matmul_body(x_ref, w_ref, b_ref, h_ref):
    h_ref[...] = lax.dot_general(
        x_ref[...], w_ref[...], (((1,), (1,)), ((), ())),
        preferred_element_type=jnp.float32,
    ) + b_ref[...]


def _linear(x, W_w, W_b):
    return pl.pallas_call(
        _matmul_body,
        grid=(5,),
        in_specs=[
            pl.BlockSpec((2000, D), lambda i: (i, 0)),
            pl.BlockSpec((D, D), lambda i: (0, 0)),
            pl.BlockSpec((1, D), lambda i: (0, 0)),
        ],
        out_specs=pl.BlockSpec((2000, D), lambda i: (i, 0)),
        out_shape=jax.ShapeDtypeStruct((N_NODES, D), jnp.float32),
    )(x, W_w, W_b.reshape(1, D))


def _combine_body(p_ref, o_ref):
    o_ref[...] = p_ref[0] + p_ref[1]


def _combine(partials):
    return pl.pallas_call(
        _combine_body,
        grid=(5,),
        in_specs=[pl.BlockSpec((NUM_CORES, 2000, D), lambda i: (0, i, 0))],
        out_specs=pl.BlockSpec((2000, D), lambda i: (i, 0)),
        out_shape=jax.ShapeDtypeStruct((N_NODES, D), jnp.float32),
    )(partials)


@functools.partial(
    pl.kernel,
    mesh=plsc.VectorSubcoreMesh(core_axis_name="c", subcore_axis_name="s"),
    out_type=jax.ShapeDtypeStruct((NUM_CORES, ACC_ROWS, D), jnp.float32),
    scratch_types=[
        pltpu.VMEM((2, CHUNK, D), jnp.float32),
        pltpu.VMEM((2, GRP * CHUNK), jnp.int32),
        pltpu.VMEM((2, GRP * CHUNK), jnp.int32),
        pltpu.VMEM((GRP, CHUNK), jnp.int32),
        pltpu.VMEM((GRP, CHUNK), jnp.int32),
        pltpu.VMEM_SHARED((ACC_ROWS, D), jnp.float32),
        pltpu.SemaphoreType.DMA,
        pltpu.SemaphoreType.DMA,
        pltpu.SemaphoreType.DMA,
    ],
)
def _sc_segment_sum(h_hbm, ei_hbm, out_hbm,
                    buf_v, col1_v, row1_v, col_v, row_v, acc_sh,
                    sem0, sem1, isem):
    cid = lax.axis_index("c")
    sid = lax.axis_index("s")
    wid = cid * NUM_SUBCORES + sid
    sems = (sem0, sem1)

    def _prefetch(first_chunk, p):
        off = pl.multiple_of(first_chunk * CHUNK, 8)
        pltpu.async_copy(
            ei_hbm.at[1, pl.ds(off, GRP * CHUNK)], col1_v.at[p], isem)
        pltpu.async_copy(
            ei_hbm.at[0, pl.ds(off, GRP * CHUNK)], row1_v.at[p], isem)

    def _drain_idx(p):
        pltpu.make_async_copy(
            ei_hbm.at[1, pl.ds(0, GRP * CHUNK)], col1_v.at[p], isem).wait()
        pltpu.make_async_copy(
            ei_hbm.at[0, pl.ds(0, GRP * CHUNK)], row1_v.at[p], isem).wait()

    def _repack(p, n_chunks):
        for k in range(n_chunks):
            for c in range(CHUNK // 16):
                col_v[k, pl.ds(c * 16, 16)] = col1_v[p, pl.ds(k * CHUNK + c * 16, 16)]
                row_v[k, pl.ds(c * 16, 16)] = row1_v[p, pl.ds(k * CHUNK + c * 16, 16)]

    n_groups = jnp.where(wid < BIG_WORKERS, GRP_BIG, GRP_SMALL)
    group_start = jnp.where(
        wid < BIG_WORKERS,
        wid * GRP_BIG,
        BIG_WORKERS * GRP_BIG + (wid - BIG_WORKERS) * GRP_SMALL,
    )

    _prefetch(group_start * GRP, 0)

    zbuf = buf_v.at[0]

    def _zrow(i, _):
        for c in range(D // 16):
            zbuf[i, pl.ds(c * 16, 16)] = jnp.zeros((16,), jnp.float32)
        return 0
    lax.fori_loop(0, CHUNK, _zrow, 0)
    for r in range(ZERO_ROWS // CHUNK):
        pltpu.sync_copy(
            zbuf, acc_sh.at[pl.ds(sid * ZERO_ROWS + r * CHUNK, CHUNK)])

    plsc.subcore_barrier()

    def _run_chunks(n_chunks):
        copies = [None, None]

        def _gather(k):
            copies[k % 2] = pltpu.async_copy(
                h_hbm.at[col_v.at[k]], buf_v.at[k % 2], sems[k % 2])

        _gather(0)
        for k in range(n_chunks):
            if k + 1 < n_chunks:
                _gather(k + 1)
            copies[k % 2].wait()
            pltpu.sync_copy(buf_v.at[k % 2], acc_sh.at[row_v.at[k]], add=True)

    def _pair(i, _):
        for p in range(2):
            g = 2 * i + p

            def _do_group(g=g, p=p):
                _drain_idx(p)

                def _issue_next(g=g, p=p):
                    _prefetch((group_start + g + 1) * GRP, 1 - p)
                pl.when(g + 1 < n_groups)(_issue_next)
                _repack(p, GRP)
                _run_chunks(GRP)
            pl.when(g < n_groups)(_do_group)
        return 0
    lax.fori_loop(0, (jnp.int32(1) + n_groups) // 2, _pair, 0)

    @pl.when(wid == NUM_WORKERS - 1)
    def _tail():
        off = pl.multiple_of(N_GROUPS * GRP * CHUNK, 8)
        pltpu.sync_copy(ei_hbm.at[1, pl.ds(off, TAIL_CHUNKS * CHUNK)],
                        col1_v.at[0, pl.ds(0, TAIL_CHUNKS * CHUNK)])
        pltpu.sync_copy(ei_hbm.at[0, pl.ds(off, TAIL_CHUNKS * CHUNK)],
                        row1_v.at[0, pl.ds(0, TAIL_CHUNKS * CHUNK)])
        _repack(0, TAIL_CHUNKS)
        _run_chunks(TAIL_CHUNKS)

    plsc.subcore_barrier()

    pltpu.sync_copy(
        acc_sh.at[pl.ds(sid * OUT_ROWS_PER_TILE, OUT_ROWS_PER_TILE)],
        out_hbm.at[cid, pl.ds(sid * OUT_ROWS_PER_TILE, OUT_ROWS_PER_TILE)],
    )


def kernel(x, edge_index, W_w, W_b, a_w, a_b):
    h = _linear(x, W_w, W_b)
    partials = _sc_segment_sum(h, edge_index.astype(jnp.int32))
    return _combine(partials)


# speedup vs baseline: 1.3282x; 1.0431x over previous
import functools

import jax
import jax.numpy as jnp
from jax import lax
from jax.experimental import pallas as pl
from jax.experimental.pallas import tpu as pltpu
from jax.experimental.pallas import tpu_sc as plsc

N_NODES = 10000
N_EDGES = 320000
D = 128

NUM_CORES = 2
NUM_SUBCORES = 16
NUM_WORKERS = NUM_CORES * NUM_SUBCORES

CHUNK = 128
GRP = 16
N_CHUNKS = N_EDGES // CHUNK
N_GROUPS = N_CHUNKS // GRP
TAIL_CHUNKS = N_CHUNKS - N_GROUPS * GRP
BIG_WORKERS = N_GROUPS % NUM_WORKERS
GRP_BIG = N_GROUPS // NUM_WORKERS + 1
GRP_SMALL = N_GROUPS // NUM_WORKERS

ACC_ROWS = 10240
ZERO_ROWS = ACC_ROWS // NUM_SUBCORES
OUT_ROWS_PER_TILE = ACC_ROWS // NUM_SUBCORES


def _matmul_body(x_ref, w_ref, b_ref, h_ref):
    h_ref[...] = lax.dot_general(
        x_ref[...], w_ref[...], (((1,), (1,)), ((), ())),
        preferred_element_type=jnp.float32,
    ) + b_ref[...]


def _linear(x, W_w, W_b):
    return pl.pallas_call(
        _matmul_body,
        grid=(5,),
        in_specs=[
            pl.BlockSpec((2000, D), lambda i: (i, 0)),
            pl.BlockSpec((D, D), lambda i: (0, 0)),
            pl.BlockSpec((1, D), lambda i: (0, 0)),
        ],
        out_specs=pl.BlockSpec((2000, D), lambda i: (i, 0)),
        out_shape=jax.ShapeDtypeStruct((N_NODES, D), jnp.float32),
    )(x, W_w, W_b.reshape(1, D))


def _combine_body(p_ref, o_ref):
    o_ref[...] = p_ref[0] + p_ref[1]


def _combine(partials):
    return pl.pallas_call(
        _combine_body,
        grid=(5,),
        in_specs=[pl.BlockSpec((NUM_CORES, 2000, D), lambda i: (0, i, 0))],
        out_specs=pl.BlockSpec((2000, D), lambda i: (i, 0)),
        out_shape=jax.ShapeDtypeStruct((N_NODES, D), jnp.float32),
    )(partials)


@functools.partial(
    pl.kernel,
    mesh=plsc.VectorSubcoreMesh(core_axis_name="c", subcore_axis_name="s"),
    out_type=jax.ShapeDtypeStruct((NUM_CORES, ACC_ROWS, D), jnp.float32),
    scratch_types=[
        pltpu.VMEM((2, CHUNK, D), jnp.float32),
        pltpu.VMEM((2, GRP * CHUNK), jnp.int32),
        pltpu.VMEM((2, GRP * CHUNK), jnp.int32),
        pltpu.VMEM((GRP, CHUNK), jnp.int32),
        pltpu.VMEM((GRP, CHUNK), jnp.int32),
        pltpu.VMEM_SHARED((ACC_ROWS, D), jnp.float32),
        pltpu.SemaphoreType.DMA,
        pltpu.SemaphoreType.DMA,
        pltpu.SemaphoreType.DMA,
    ],
)
def _sc_segment_sum(h_hbm, ei_hbm, out_hbm,
                    buf_v, col1_v, row1_v, col_v, row_v, acc_sh,
                    sem0, sem1, isem):
    cid = lax.axis_index("c")
    sid = lax.axis_index("s")
    wid = cid * NUM_SUBCORES + sid
    sems = (sem0, sem1)

    def _prefetch(first_chunk, p):
        off = pl.multiple_of(first_chunk * CHUNK, 8)
        pltpu.async_copy(
            ei_hbm.at[1, pl.ds(off, GRP * CHUNK)], col1_v.at[p], isem)
        pltpu.async_copy(
            ei_hbm.at[0, pl.ds(off, GRP * CHUNK)], row1_v.at[p], isem)

    def _drain_idx(p):
        pltpu.make_async_copy(
            ei_hbm.at[1, pl.ds(0, GRP * CHUNK)], col1_v.at[p], isem).wait()
        pltpu.make_async_copy(
            ei_hbm.at[0, pl.ds(0, GRP * CHUNK)], row1_v.at[p], isem).wait()

    def _repack(p, n_chunks):
        for k in range(n_chunks):
            for c in range(CHUNK // 16):
                col_v[k, pl.ds(c * 16, 16)] = col1_v[p, pl.ds(k * CHUNK + c * 16, 16)]
                row_v[k, pl.ds(c * 16, 16)] = row1_v[p, pl.ds(k * CHUNK + c * 16, 16)]

    n_groups = jnp.where(wid < BIG_WORKERS, GRP_BIG, GRP_SMALL)
    group_start = jnp.where(
        wid < BIG_WORKERS,
        wid * GRP_BIG,
        BIG_WORKERS * GRP_BIG + (wid - BIG_WORKERS) * GRP_SMALL,
    )

    _prefetch(group_start * GRP, 0)

    zbuf = buf_v.at[0]

    def _zrow(i, _):
        for c in range(D // 16):
            zbuf[i, pl.ds(c * 16, 16)] = jnp.zeros((16,), jnp.float32)
        return 0
    lax.fori_loop(0, CHUNK, _zrow, 0)
    for r in range(ZERO_ROWS // CHUNK):
        pltpu.sync_copy(
            zbuf, acc_sh.at[pl.ds(sid * ZERO_ROWS + r * CHUNK, CHUNK)])

    plsc.subcore_barrier()

    def _run_chunks(n_chunks):
        copies = [None, None]

        def _gather(k):
            copies[k % 2] = pltpu.async_copy(
                h_hbm.at[col_v.at[k]], buf_v.at[k % 2], sems[k % 2])

        _gather(0)
        for k in range(n_chunks):
            if k + 1 < n_chunks:
                _gather(k + 1)
            copies[k % 2].wait()
            pltpu.sync_copy(buf_v.at[k % 2], acc_sh.at[row_v.at[k]], add=True)

    def _pair(i, _):
        for p in range(2):
            g = 2 * i + p

            def _do_group(g=g, p=p):
                _drain_idx(p)

                def _issue_next(g=g, p=p):
                    _prefetch((group_start + g + 1) * GRP, 1 - p)
                pl.when(g + 1 < n_groups)(_issue_next)
                _repack(p, GRP)
                _run_chunks(GRP)
            pl.when(g < n_groups)(_do_group)
        return 0
    lax.fori_loop(0, (jnp.int32(1) + n_groups) // 2, _pair, 0)

    @pl.when(wid == NUM_WORKERS - 1)
    def _tail():
        off = pl.multiple_of(N_GROUPS * GRP * CHUNK, 8)
        pltpu.sync_copy(ei_hbm.at[1, pl.ds(off, TAIL_CHUNKS * CHUNK)],
                        col1_v.at[0, pl.ds(0, TAIL_CHUNKS * CHUNK)])
        pltpu.sync_copy(ei_hbm.at[0, pl.ds(off, TAIL_CHUNKS * CHUNK)],
                        row1_v.at[0, pl.ds(0, TAIL_CHUNKS * CHUNK)])
        _repack(0, TAIL_CHUNKS)
        _run_chunks(TAIL_CHUNKS)

    plsc.subcore_barrier()

    pltpu.sync_copy(
        acc_sh.at[pl.ds(sid * OUT_ROWS_PER_TILE, OUT_ROWS_PER_TILE)],
        out_hbm.at[cid, pl.ds(sid * OUT_ROWS_PER_TILE, OUT_ROWS_PER_TILE)],
    )


def kernel(x, edge_index, W_w, W_b, a_w, a_b):
    h = _linear(x, W_w, W_b)
    partials = _sc_segment_sum(h, edge_index.astype(jnp.int32))
    return _combine(partials)


# speedup vs baseline: 1.3538x; 1.0193x over previous
import functools

import jax
import jax.numpy as jnp
from jax import lax
from jax.experimental import pallas as pl
from jax.experimental.pallas import tpu as pltpu
from jax.experimental.pallas import tpu_sc as plsc

N_NODES = 10000
N_EDGES = 320000
D = 128

NUM_CORES = 2
NUM_SUBCORES = 16
NUM_WORKERS = NUM_CORES * NUM_SUBCORES

CHUNK = 128
GRP = 20
N_CHUNKS = N_EDGES // CHUNK
N_GROUPS = N_CHUNKS // GRP
TAIL_CHUNKS = N_CHUNKS - N_GROUPS * GRP
BIG_WORKERS = N_GROUPS % NUM_WORKERS
GRP_BIG = N_GROUPS // NUM_WORKERS + 1
GRP_SMALL = N_GROUPS // NUM_WORKERS

ACC_ROWS = 10240
ZERO_ROWS = ACC_ROWS // NUM_SUBCORES
OUT_ROWS_PER_TILE = ACC_ROWS // NUM_SUBCORES


def _matmul_body(x_ref, w_ref, b_ref, h_ref):
    h_ref[...] = lax.dot_general(
        x_ref[...], w_ref[...], (((1,), (1,)), ((), ())),
        preferred_element_type=jnp.float32,
    ) + b_ref[...]


def _linear(x, W_w, W_b):
    return pl.pallas_call(
        _matmul_body,
        grid=(2,),
        in_specs=[
            pl.BlockSpec((5000, D), lambda i: (i, 0)),
            pl.BlockSpec((D, D), lambda i: (0, 0)),
            pl.BlockSpec((1, D), lambda i: (0, 0)),
        ],
        out_specs=pl.BlockSpec((5000, D), lambda i: (i, 0)),
        out_shape=jax.ShapeDtypeStruct((N_NODES, D), jnp.float32),
    )(x, W_w, W_b.reshape(1, D))


def _combine_body(p_ref, o_ref):
    o_ref[...] = p_ref[0] + p_ref[1]


def _combine(partials):
    return pl.pallas_call(
        _combine_body,
        grid=(5,),
        in_specs=[pl.BlockSpec((NUM_CORES, 2000, D), lambda i: (0, i, 0))],
        out_specs=pl.BlockSpec((2000, D), lambda i: (i, 0)),
        out_shape=jax.ShapeDtypeStruct((N_NODES, D), jnp.float32),
    )(partials)


@functools.partial(
    pl.kernel,
    mesh=plsc.VectorSubcoreMesh(core_axis_name="c", subcore_axis_name="s"),
    out_type=jax.ShapeDtypeStruct((NUM_CORES, ACC_ROWS, D), jnp.float32),
    scratch_types=[
        pltpu.VMEM((2, CHUNK, D), jnp.float32),
        pltpu.VMEM((2, GRP * CHUNK), jnp.int32),
        pltpu.VMEM((2, GRP * CHUNK), jnp.int32),
        pltpu.VMEM((GRP, CHUNK), jnp.int32),
        pltpu.VMEM((GRP, CHUNK), jnp.int32),
        pltpu.VMEM_SHARED((ACC_ROWS, D), jnp.float32),
        pltpu.SemaphoreType.DMA,
        pltpu.SemaphoreType.DMA,
        pltpu.SemaphoreType.DMA,
    ],
)
def _sc_segment_sum(h_hbm, ei_hbm, out_hbm,
                    buf_v, col1_v, row1_v, col_v, row_v, acc_sh,
                    sem0, sem1, isem):
    cid = lax.axis_index("c")
    sid = lax.axis_index("s")
    wid = cid * NUM_SUBCORES + sid
    sems = (sem0, sem1)

    def _prefetch(first_chunk, p):
        off = pl.multiple_of(first_chunk * CHUNK, 8)
        pltpu.async_copy(
            ei_hbm.at[1, pl.ds(off, GRP * CHUNK)], col1_v.at[p], isem)
        pltpu.async_copy(
            ei_hbm.at[0, pl.ds(off, GRP * CHUNK)], row1_v.at[p], isem)

    def _drain_idx(p):
        pltpu.make_async_copy(
            ei_hbm.at[1, pl.ds(0, GRP * CHUNK)], col1_v.at[p], isem).wait()
        pltpu.make_async_copy(
            ei_hbm.at[0, pl.ds(0, GRP * CHUNK)], row1_v.at[p], isem).wait()

    def _repack(p, n_chunks):
        for k in range(n_chunks):
            for c in range(CHUNK // 16):
                col_v[k, pl.ds(c * 16, 16)] = col1_v[p, pl.ds(k * CHUNK + c * 16, 16)]
                row_v[k, pl.ds(c * 16, 16)] = row1_v[p, pl.ds(k * CHUNK + c * 16, 16)]

    n_groups = jnp.where(wid < BIG_WORKERS, GRP_BIG, GRP_SMALL)
    group_start = jnp.where(
        wid < BIG_WORKERS,
        wid * GRP_BIG,
        BIG_WORKERS * GRP_BIG + (wid - BIG_WORKERS) * GRP_SMALL,
    )

    _prefetch(group_start * GRP, 0)

    zbuf = buf_v.at[0]

    def _zrow(i, _):
        for c in range(D // 16):
            zbuf[i, pl.ds(c * 16, 16)] = jnp.zeros((16,), jnp.float32)
        return 0
    lax.fori_loop(0, CHUNK, _zrow, 0)
    for r in range(ZERO_ROWS // CHUNK):
        pltpu.sync_copy(
            zbuf, acc_sh.at[pl.ds(sid * ZERO_ROWS + r * CHUNK, CHUNK)])

    plsc.subcore_barrier()

    def _run_chunks(n_chunks):
        copies = [None, None]

        def _gather(k):
            copies[k % 2] = pltpu.async_copy(
                h_hbm.at[col_v.at[k]], buf_v.at[k % 2], sems[k % 2])

        _gather(0)
        for k in range(n_chunks):
            if k + 1 < n_chunks:
                _gather(k + 1)
            copies[k % 2].wait()
            pltpu.sync_copy(buf_v.at[k % 2], acc_sh.at[row_v.at[k]], add=True)

    def _pair(i, _):
        for p in range(2):
            g = 2 * i + p

            def _do_group(g=g, p=p):
                _drain_idx(p)

                def _issue_next(g=g, p=p):
                    _prefetch((group_start + g + 1) * GRP, 1 - p)
                pl.when(g + 1 < n_groups)(_issue_next)
                _repack(p, GRP)
                _run_chunks(GRP)
            pl.when(g < n_groups)(_do_group)
        return 0
    lax.fori_loop(0, (jnp.int32(1) + n_groups) // 2, _pair, 0)

    def _tail():
        off = pl.multiple_of(N_GROUPS * GRP * CHUNK, 8)
        pltpu.sync_copy(ei_hbm.at[1, pl.ds(off, TAIL_CHUNKS * CHUNK)],
                        col1_v.at[0, pl.ds(0, TAIL_CHUNKS * CHUNK)])
        pltpu.sync_copy(ei_hbm.at[0, pl.ds(off, TAIL_CHUNKS * CHUNK)],
                        row1_v.at[0, pl.ds(0, TAIL_CHUNKS * CHUNK)])
        _repack(0, TAIL_CHUNKS)
        _run_chunks(TAIL_CHUNKS)

    if TAIL_CHUNKS:
        pl.when(wid == NUM_WORKERS - 1)(_tail)

    plsc.subcore_barrier()

    pltpu.sync_copy(
        acc_sh.at[pl.ds(sid * OUT_ROWS_PER_TILE, OUT_ROWS_PER_TILE)],
        out_hbm.at[cid, pl.ds(sid * OUT_ROWS_PER_TILE, OUT_ROWS_PER_TILE)],
    )


def kernel(x, edge_index, W_w, W_b, a_w, a_b):
    h = _linear(x, W_w, W_b)
    partials = _sc_segment_sum(h, edge_index.astype(jnp.int32))
    return _combine(partials)


# speedup vs baseline: 1.3748x; 1.0155x over previous
import functools

import jax
import jax.numpy as jnp
from jax import lax
from jax.experimental import pallas as pl
from jax.experimental.pallas import tpu as pltpu
from jax.experimental.pallas import tpu_sc as plsc

N_NODES = 10000
N_EDGES = 320000
D = 128

NUM_CORES = 2
NUM_SUBCORES = 16
NUM_WORKERS = NUM_CORES * NUM_SUBCORES

CHUNK = 128
GRP = 20
N_CHUNKS = N_EDGES // CHUNK
N_GROUPS = N_CHUNKS // GRP
TAIL_CHUNKS = N_CHUNKS - N_GROUPS * GRP
BIG_WORKERS = N_GROUPS % NUM_WORKERS
GRP_BIG = N_GROUPS // NUM_WORKERS + 1
GRP_SMALL = N_GROUPS // NUM_WORKERS

ACC_ROWS = 10240
ZERO_ROWS = ACC_ROWS // NUM_SUBCORES
OUT_ROWS_PER_TILE = ACC_ROWS // NUM_SUBCORES


def _matmul_body(x_ref, w_ref, b_ref, h_ref):
    h_ref[...] = lax.dot_general(
        x_ref[...], w_ref[...], (((1,), (1,)), ((), ())),
        preferred_element_type=jnp.float32,
    ) + b_ref[...]


def _linear(x, W_w, W_b):
    return pl.pallas_call(
        _matmul_body,
        grid=(2,),
        in_specs=[
            pl.BlockSpec((5000, D), lambda i: (i, 0)),
            pl.BlockSpec((D, D), lambda i: (0, 0)),
            pl.BlockSpec((1, D), lambda i: (0, 0)),
        ],
        out_specs=pl.BlockSpec((5000, D), lambda i: (i, 0)),
        out_shape=jax.ShapeDtypeStruct((N_NODES, D), jnp.float32),
    )(x, W_w, W_b.reshape(1, D))


def _combine_body(p_ref, o_ref):
    o_ref[...] = p_ref[0] + p_ref[1]


def _combine(partials):
    return pl.pallas_call(
        _combine_body,
        grid=(2,),
        in_specs=[pl.BlockSpec((NUM_CORES, 5000, D), lambda i: (0, i, 0))],
        out_specs=pl.BlockSpec((5000, D), lambda i: (i, 0)),
        out_shape=jax.ShapeDtypeStruct((N_NODES, D), jnp.float32),
    )(partials)


@functools.partial(
    pl.kernel,
    mesh=plsc.VectorSubcoreMesh(core_axis_name="c", subcore_axis_name="s"),
    out_type=jax.ShapeDtypeStruct((NUM_CORES, ACC_ROWS, D), jnp.float32),
    scratch_types=[
        pltpu.VMEM((2, CHUNK, D), jnp.float32),
        pltpu.VMEM((2, GRP * CHUNK), jnp.int32),
        pltpu.VMEM((2, GRP * CHUNK), jnp.int32),
        pltpu.VMEM((GRP, CHUNK), jnp.int32),
        pltpu.VMEM((GRP, CHUNK), jnp.int32),
        pltpu.VMEM_SHARED((ACC_ROWS, D), jnp.float32),
        pltpu.SemaphoreType.DMA,
        pltpu.SemaphoreType.DMA,
        pltpu.SemaphoreType.DMA,
    ],
)
def _sc_segment_sum(h_hbm, ei_hbm, out_hbm,
                    buf_v, col1_v, row1_v, col_v, row_v, acc_sh,
                    sem0, sem1, isem):
    cid = lax.axis_index("c")
    sid = lax.axis_index("s")
    wid = cid * NUM_SUBCORES + sid
    sems = (sem0, sem1)

    def _prefetch(first_chunk, p):
        off = pl.multiple_of(first_chunk * CHUNK, 8)
        pltpu.async_copy(
            ei_hbm.at[1, pl.ds(off, GRP * CHUNK)], col1_v.at[p], isem)
        pltpu.async_copy(
            ei_hbm.at[0, pl.ds(off, GRP * CHUNK)], row1_v.at[p], isem)

    def _drain_idx(p):
        pltpu.make_async_copy(
            ei_hbm.at[1, pl.ds(0, GRP * CHUNK)], col1_v.at[p], isem).wait()
        pltpu.make_async_copy(
            ei_hbm.at[0, pl.ds(0, GRP * CHUNK)], row1_v.at[p], isem).wait()

    def _repack(p, n_chunks):
        for k in range(n_chunks):
            for c in range(CHUNK // 16):
                col_v[k, pl.ds(c * 16, 16)] = col1_v[p, pl.ds(k * CHUNK + c * 16, 16)]
                row_v[k, pl.ds(c * 16, 16)] = row1_v[p, pl.ds(k * CHUNK + c * 16, 16)]

    n_groups = jnp.where(wid < BIG_WORKERS, GRP_BIG, GRP_SMALL)
    group_start = jnp.where(
        wid < BIG_WORKERS,
        wid * GRP_BIG,
        BIG_WORKERS * GRP_BIG + (wid - BIG_WORKERS) * GRP_SMALL,
    )

    _prefetch(group_start * GRP, 0)

    zbuf = buf_v.at[0]

    def _zrow(i, _):
        for c in range(D // 16):
            zbuf[i, pl.ds(c * 16, 16)] = jnp.zeros((16,), jnp.float32)
        return 0
    lax.fori_loop(0, CHUNK, _zrow, 0)
    for r in range(ZERO_ROWS // CHUNK):
        pltpu.sync_copy(
            zbuf, acc_sh.at[pl.ds(sid * ZERO_ROWS + r * CHUNK, CHUNK)])

    plsc.subcore_barrier()

    def _run_chunks(n_chunks):
        copies = [None, None]

        def _gather(k):
            copies[k % 2] = pltpu.async_copy(
                h_hbm.at[col_v.at[k]], buf_v.at[k % 2], sems[k % 2])

        _gather(0)
        for k in range(n_chunks):
            if k + 1 < n_chunks:
                _gather(k + 1)
            copies[k % 2].wait()
            pltpu.sync_copy(buf_v.at[k % 2], acc_sh.at[row_v.at[k]], add=True)

    def _pair(i, _):
        for p in range(2):
            g = 2 * i + p

            def _do_group(g=g, p=p):
                _drain_idx(p)

                def _issue_next(g=g, p=p):
                    _prefetch((group_start + g + 1) * GRP, 1 - p)
                pl.when(g + 1 < n_groups)(_issue_next)
                _repack(p, GRP)
                _run_chunks(GRP)
            pl.when(g < n_groups)(_do_group)
        return 0
    lax.fori_loop(0, (jnp.int32(1) + n_groups) // 2, _pair, 0)

    def _tail():
        off = pl.multiple_of(N_GROUPS * GRP * CHUNK, 8)
        pltpu.sync_copy(ei_hbm.at[1, pl.ds(off, TAIL_CHUNKS * CHUNK)],
                        col1_v.at[0, pl.ds(0, TAIL_CHUNKS * CHUNK)])
        pltpu.sync_copy(ei_hbm.at[0, pl.ds(off, TAIL_CHUNKS * CHUNK)],
                        row1_v.at[0, pl.ds(0, TAIL_CHUNKS * CHUNK)])
        _repack(0, TAIL_CHUNKS)
        _run_chunks(TAIL_CHUNKS)

    if TAIL_CHUNKS:
        pl.when(wid == NUM_WORKERS - 1)(_tail)

    plsc.subcore_barrier()

    pltpu.sync_copy(
        acc_sh.at[pl.ds(sid * OUT_ROWS_PER_TILE, OUT_ROWS_PER_TILE)],
        out_hbm.at[cid, pl.ds(sid * OUT_ROWS_PER_TILE, OUT_ROWS_PER_TILE)],
    )


def kernel(x, edge_index, W_w, W_b, a_w, a_b):
    h = _linear(x, W_w, W_b)
    partials = _sc_segment_sum(h, edge_index.astype(jnp.int32))
    return _combine(partials)
